# baseline (speedup 1.0000x reference)
import functools

import jax
import jax.numpy as jnp
from jax import lax
from jax.experimental import pallas as pl
from jax.experimental.pallas import tpu as pltpu
from jax.experimental.pallas import tpu_sc as plsc

N_CAT = 238
D_IMG = 3 * 32 * 32
BM = 512


BN = 512


def _mm_body(wt_ref, xt_ref, bt_ref, logpt_ref, ent_ref, kl_ref):
    wt = wt_ref[...]
    xt = xt_ref[...]
    logits = jnp.dot(wt, xt, preferred_element_type=jnp.float32) + bt_ref[...]
    m = jnp.max(logits, axis=0, keepdims=True)
    sh = logits - m
    lse = jnp.log(jnp.sum(jnp.exp(sh), axis=0, keepdims=True))
    logp = sh - lse
    p = jnp.exp(logp)
    logpt_ref[...] = logp
    ent_ref[...] = -jnp.sum(p * logp, axis=0, keepdims=True)
    kl_ref[...] = jnp.sum(p * (logp - jnp.log(1.0 / N_CAT)), axis=0,
                          keepdims=True)


def _mm_call(wt, xt, bt):
    bsz = xt.shape[1]
    grid = (bsz // BN,)
    return pl.pallas_call(
        _mm_body,
        grid=grid,
        in_specs=[
            pl.BlockSpec((N_CAT, D_IMG), lambda i: (0, 0)),
            pl.BlockSpec((D_IMG, BN), lambda i: (0, i)),
            pl.BlockSpec((N_CAT, 1), lambda i: (0, 0)),
        ],
        out_specs=[
            pl.BlockSpec((N_CAT, BN), lambda i: (0, i)),
            pl.BlockSpec((1, BN), lambda i: (0, i)),
            pl.BlockSpec((1, BN), lambda i: (0, i)),
        ],
        out_shape=[
            jax.ShapeDtypeStruct((N_CAT, bsz), jnp.float32),
            jax.ShapeDtypeStruct((1, bsz), jnp.float32),
            jax.ShapeDtypeStruct((1, bsz), jnp.float32),
        ],
        compiler_params=pltpu.CompilerParams(
            dimension_semantics=("parallel",),
        ),
    )(wt, xt, bt)


def _samp_body(logp_ref, u_ref, samp_ref, slp_ref):
    n_copies = u_ref.shape[0]
    logp = logp_ref[...]
    iota = lax.broadcasted_iota(jnp.int32, (BM, N_CAT), 1)
    for k in range(n_copies):
        g = -jnp.log(-jnp.log(u_ref[k]))
        s = jnp.argmax(logp + g, axis=-1).astype(jnp.int32)
        samp_ref[k, :] = s
        slp_ref[k, :] = jnp.sum(jnp.where(iota == s[:, None], logp, 0.0), axis=-1)


def _samp_call(logp_row, u):
    bsz = logp_row.shape[0]
    n_copies = u.shape[0]
    grid = (bsz // BM,)
    return pl.pallas_call(
        _samp_body,
        grid=grid,
        in_specs=[
            pl.BlockSpec((BM, N_CAT), lambda i: (i, 0)),
            pl.BlockSpec((n_copies, BM, N_CAT), lambda i: (0, i, 0)),
        ],
        out_specs=[
            pl.BlockSpec((n_copies, BM), lambda i: (0, i)),
            pl.BlockSpec((n_copies, BM), lambda i: (0, i)),
        ],
        out_shape=[
            jax.ShapeDtypeStruct((n_copies, bsz), jnp.int32),
            jax.ShapeDtypeStruct((n_copies, bsz), jnp.float32),
        ],
        compiler_params=pltpu.CompilerParams(
            dimension_semantics=("parallel",),
        ),
    )(logp_row, u)


F_TC = 2048
F_SC = D_IMG - F_TC
_SCH = 1024


def _sc_tgather_body(f0, pw_f, n_rows, idx_hbm, bank_t_hbm, out_hbm,
                     idx_v, table_v, rows0, rows1, ss0, ss1):
    wid = lax.axis_index("s") * 2 + lax.axis_index("c")
    fbase = f0 + wid * pw_f
    obase = wid * pw_f
    pltpu.sync_copy(idx_hbm, idx_v)
    pltpu.sync_copy(bank_t_hbm.at[pl.ds(fbase, pw_f)], table_v)
    bufs = ((rows0, ss0), (rows1, ss1))
    n_ch = n_rows // _SCH
    for c in range(n_ch):
        buf, ssem = bufs[c % 2]
        if c >= 2:
            pltpu.make_async_copy(
                buf, out_hbm.at[pl.ds(obase, pw_f), pl.ds(c * _SCH, _SCH)],
                ssem,
            ).wait()

        def body(v, carry, c=c, buf=buf):
            sv = idx_v[pl.ds(c * _SCH + v * 16, 16)]
            for f in range(pw_f):
                fv = jnp.full((16,), f, jnp.int32)
                buf[f, pl.ds(v * 16, 16)] = plsc.load_gather(table_v, [fv, sv])
            return carry

        lax.fori_loop(0, _SCH // 16, body, 0)
        pltpu.async_copy(
            buf, out_hbm.at[pl.ds(obase, pw_f), pl.ds(c * _SCH, _SCH)], ssem
        )
    for c in (n_ch - 2, n_ch - 1):
        buf, ssem = bufs[c % 2]
        pltpu.make_async_copy(
            buf, out_hbm.at[pl.ds(obase, pw_f), pl.ds(c * _SCH, _SCH)], ssem
        ).wait()


def _sc_tgather_call(idx_flat, bank_t):
    n_rows = idx_flat.shape[0]
    info = plsc.get_sparse_core_info()
    nw = info.num_cores * info.num_subcores
    pw_f = F_SC // nw
    mesh = plsc.VectorSubcoreMesh(core_axis_name="c", subcore_axis_name="s")
    kfn = pl.kernel(
        functools.partial(_sc_tgather_body, F_TC, pw_f, n_rows),
        mesh=mesh,
        out_type=jax.ShapeDtypeStruct((F_SC, n_rows), jnp.float32),
        scratch_types=[
            pltpu.VMEM((n_rows,), jnp.int32),
            pltpu.VMEM((pw_f, N_CAT), jnp.float32),
            pltpu.VMEM((pw_f, _SCH), jnp.float32),
            pltpu.VMEM((pw_f, _SCH), jnp.float32),
            pltpu.SemaphoreType.DMA,
            pltpu.SemaphoreType.DMA,
        ],
        compiler_params=pltpu.CompilerParams(needs_layout_passes=False),
    )
    return kfn(idx_flat, bank_t)


_BF = 512
_BS = 2048


def _onehot_body(bank_t_ref, samp_ref, out_ref):
    oh = (lax.broadcasted_iota(jnp.int32, (N_CAT, _BS), 0)
          == samp_ref[...]).astype(jnp.float32)
    out_ref[...] = jnp.dot(bank_t_ref[...], oh,
                           preferred_element_type=jnp.float32)


def _onehot_gather_call(bank_t, idx_row, n_rows, f_rows):
    grid = (f_rows // _BF, n_rows // _BS)
    return pl.pallas_call(
        _onehot_body,
        grid=grid,
        in_specs=[
            pl.BlockSpec((_BF, N_CAT), lambda fi, si: (fi, 0)),
            pl.BlockSpec((1, _BS), lambda fi, si: (0, si)),
        ],
        out_specs=pl.BlockSpec((_BF, _BS), lambda fi, si: (fi, si)),
        out_shape=jax.ShapeDtypeStruct((f_rows, n_rows), jnp.float32),
        compiler_params=pltpu.CompilerParams(
            dimension_semantics=("parallel", "parallel"),
        ),
    )(bank_t, idx_row)


def kernel(x, u, W, b, bank, n_copies):
    bsz = x.shape[0]
    n_copies_static = u.shape[0]
    n_rows = n_copies_static * bsz
    xt = x.reshape(bsz, -1).T
    logpt, ent, kl = _mm_call(W.T, xt, b.reshape(-1, 1))
    samp, slp = _samp_call(logpt.T, u)
    idx_row = samp.reshape(1, n_rows)
    bank_t = bank.reshape(N_CAT, D_IMG).T
    out_sc = _sc_tgather_call(samp.reshape(-1), bank_t)
    out_tc = _onehot_gather_call(bank_t, idx_row, n_rows, F_TC)
    c, h, w = bank.shape[1:]
    c_tc = F_TC // (h * w)
    xo_tc = out_tc.reshape(c_tc, h, w, n_rows).transpose(3, 0, 1, 2)
    xo_sc = out_sc.reshape(c - c_tc, h, w, n_rows).transpose(3, 0, 1, 2)
    x_out = jax.lax.stop_gradient(jnp.concatenate([xo_tc, xo_sc], axis=1))
    return (x_out, slp.reshape(-1), ent.reshape(-1), kl.reshape(-1))


# speedup vs baseline: 2.6272x; 2.6272x over previous
import jax
import jax.numpy as jnp
from jax import lax
from jax.experimental import pallas as pl
from jax.experimental.pallas import tpu as pltpu

N_CAT = 238
D_IMG = 3 * 32 * 32
BM = 512
BN = 512


def _mm_body(wt_ref, xt_ref, bt_ref, logpt_ref, ent_ref, kl_ref):
    wt = wt_ref[...]
    xt = xt_ref[...]
    logits = jnp.dot(wt, xt, preferred_element_type=jnp.float32) + bt_ref[...]
    m = jnp.max(logits, axis=0, keepdims=True)
    sh = logits - m
    lse = jnp.log(jnp.sum(jnp.exp(sh), axis=0, keepdims=True))
    logp = sh - lse
    p = jnp.exp(logp)
    logpt_ref[...] = logp
    ent_ref[...] = -jnp.sum(p * logp, axis=0, keepdims=True)
    kl_ref[...] = jnp.sum(p * (logp - jnp.log(1.0 / N_CAT)), axis=0,
                          keepdims=True)


def _mm_call(wt, xt, bt):
    bsz = xt.shape[1]
    grid = (bsz // BN,)
    return pl.pallas_call(
        _mm_body,
        grid=grid,
        in_specs=[
            pl.BlockSpec((N_CAT, D_IMG), lambda i: (0, 0)),
            pl.BlockSpec((D_IMG, BN), lambda i: (0, i)),
            pl.BlockSpec((N_CAT, 1), lambda i: (0, 0)),
        ],
        out_specs=[
            pl.BlockSpec((N_CAT, BN), lambda i: (0, i)),
            pl.BlockSpec((1, BN), lambda i: (0, i)),
            pl.BlockSpec((1, BN), lambda i: (0, i)),
        ],
        out_shape=[
            jax.ShapeDtypeStruct((N_CAT, bsz), jnp.float32),
            jax.ShapeDtypeStruct((1, bsz), jnp.float32),
            jax.ShapeDtypeStruct((1, bsz), jnp.float32),
        ],
        compiler_params=pltpu.CompilerParams(
            dimension_semantics=("parallel",),
        ),
    )(wt, xt, bt)


def _samp_body(logp_ref, u_ref, samp_ref, slp_ref):
    n_copies = u_ref.shape[0]
    logp = logp_ref[...]
    iota = lax.broadcasted_iota(jnp.int32, (BM, N_CAT), 1)
    for k in range(n_copies):
        g = -jnp.log(-jnp.log(u_ref[k]))
        s = jnp.argmax(logp + g, axis=-1).astype(jnp.int32)
        samp_ref[k, :] = s
        slp_ref[k, :] = jnp.sum(jnp.where(iota == s[:, None], logp, 0.0), axis=-1)


def _samp_call(logp_row, u):
    bsz = logp_row.shape[0]
    n_copies = u.shape[0]
    grid = (bsz // BM,)
    return pl.pallas_call(
        _samp_body,
        grid=grid,
        in_specs=[
            pl.BlockSpec((BM, N_CAT), lambda i: (i, 0)),
            pl.BlockSpec((n_copies, BM, N_CAT), lambda i: (0, i, 0)),
        ],
        out_specs=[
            pl.BlockSpec((n_copies, BM), lambda i: (0, i)),
            pl.BlockSpec((n_copies, BM), lambda i: (0, i)),
        ],
        out_shape=[
            jax.ShapeDtypeStruct((n_copies, bsz), jnp.int32),
            jax.ShapeDtypeStruct((n_copies, bsz), jnp.float32),
        ],
        compiler_params=pltpu.CompilerParams(
            dimension_semantics=("parallel",),
        ),
    )(logp_row, u)


_BF = 512
_BS = 4096


def _onehot_body(bank_t_ref, samp_ref, out_ref):
    oh = (lax.broadcasted_iota(jnp.int32, (N_CAT, _BS), 0)
          == samp_ref[...]).astype(jnp.float32)
    out_ref[...] = jnp.dot(bank_t_ref[...], oh,
                           preferred_element_type=jnp.float32)


def _onehot_gather_call(bank_t, idx_row, n_rows, f_rows):
    grid = (f_rows // _BF, n_rows // _BS)
    return pl.pallas_call(
        _onehot_body,
        grid=grid,
        in_specs=[
            pl.BlockSpec((_BF, N_CAT), lambda fi, si: (fi, 0)),
            pl.BlockSpec((1, _BS), lambda fi, si: (0, si)),
        ],
        out_specs=pl.BlockSpec((_BF, _BS), lambda fi, si: (fi, si)),
        out_shape=jax.ShapeDtypeStruct((f_rows, n_rows), jnp.float32),
        compiler_params=pltpu.CompilerParams(
            dimension_semantics=("parallel", "parallel"),
        ),
    )(bank_t, idx_row)


def kernel(x, u, W, b, bank, n_copies):
    bsz = x.shape[0]
    n_copies_static = u.shape[0]
    n_rows = n_copies_static * bsz
    xt = x.reshape(bsz, -1).T
    logpt, ent, kl = _mm_call(W.T, xt, b.reshape(-1, 1))
    samp, slp = _samp_call(logpt.T, u)
    idx_row = samp.reshape(1, n_rows)
    bank_t = bank.reshape(N_CAT, D_IMG).T
    out_t = _onehot_gather_call(bank_t, idx_row, n_rows, D_IMG)
    c, h, w = bank.shape[1:]
    x_out = jax.lax.stop_gradient(
        out_t.reshape(c, h, w, n_rows).transpose(3, 0, 1, 2)
    )
    return (x_out, slp.reshape(-1), ent.reshape(-1), kl.reshape(-1))
